# Initial kernel scaffold; baseline (speedup 1.0000x reference)
import functools
import jax
import jax.numpy as jnp
from jax import lax
from jax.experimental import pallas as pl
from jax.experimental.pallas import tpu as pltpu
from jax.experimental.pallas import tpu_sc as plsc

N = 10000
D = 128
E = 320000

NC = 2
NS = 16
NW = NC * NS
CHUNK = 128
E_PAD = 327680
EPW = E_PAD // NW
N_CHUNKS = EPW // CHUNK
N_PAD = 10016
ROWS_PER_TILE_INIT = N_PAD // NS
ROWS_PER_TILE_OUT = N // NS

_sc_mesh = plsc.VectorSubcoreMesh(core_axis_name="c", subcore_axis_name="s")


@functools.partial(
    pl.kernel,
    out_type=jax.ShapeDtypeStruct((NC, N, D), jnp.float32),
    mesh=_sc_mesh,
    scratch_types=[
        pltpu.VMEM((CHUNK,), jnp.int32),
        pltpu.VMEM((CHUNK,), jnp.int32),
        pltpu.VMEM((CHUNK, D), jnp.float32),
        pltpu.VMEM_SHARED((N_PAD, D), jnp.float32),
        pltpu.SemaphoreType.DMA,
    ],
)
def _sc_segment_sum(g_hbm, src_hbm, dst_hbm, zeros_hbm, out_hbm,
                    src_v, dst_v, rows_v, acc, sem):
    c = lax.axis_index("c")
    s = lax.axis_index("s")
    wid = s * NC + c

    pltpu.sync_copy(zeros_hbm, acc.at[pl.ds(s * ROWS_PER_TILE_INIT,
                                            ROWS_PER_TILE_INIT)])
    plsc.subcore_barrier()

    base = wid * EPW

    @pl.loop(0, N_CHUNKS)
    def _(j):
        off = base + j * CHUNK
        pltpu.sync_copy(src_hbm.at[pl.ds(off, CHUNK)], src_v)
        pltpu.sync_copy(dst_hbm.at[pl.ds(off, CHUNK)], dst_v)
        pltpu.async_copy(g_hbm.at[src_v], rows_v, sem).wait()
        pltpu.sync_copy(rows_v, acc.at[dst_v], add=True)

    plsc.subcore_barrier()

    pltpu.sync_copy(acc.at[pl.ds(s * ROWS_PER_TILE_OUT, ROWS_PER_TILE_OUT)],
                    out_hbm.at[c, pl.ds(s * ROWS_PER_TILE_OUT, ROWS_PER_TILE_OUT)])


ROW_BLK = 1000


def _tc_head(h, wlT, wcT, bias):
    def body(h_ref, wl_ref, wc_ref, b_ref, g_ref, d_ref):
        hb = h_ref[...]
        g_ref[...] = jnp.dot(hb, wl_ref[...], preferred_element_type=jnp.float32)
        d_ref[...] = jnp.dot(hb, wc_ref[...], preferred_element_type=jnp.float32) + b_ref[...]

    return pl.pallas_call(
        body,
        grid=(N // ROW_BLK,),
        in_specs=[
            pl.BlockSpec((ROW_BLK, D), lambda i: (i, 0)),
            pl.BlockSpec((D, D), lambda i: (0, 0)),
            pl.BlockSpec((D, D), lambda i: (0, 0)),
            pl.BlockSpec((1, D), lambda i: (0, 0)),
        ],
        out_specs=[
            pl.BlockSpec((ROW_BLK, D), lambda i: (i, 0)),
            pl.BlockSpec((ROW_BLK, D), lambda i: (i, 0)),
        ],
        out_shape=[
            jax.ShapeDtypeStruct((N, D), jnp.float32),
            jax.ShapeDtypeStruct((N, D), jnp.float32),
        ],
    )(h, wlT, wcT, bias)


def _tc_mid(p, d, wlT, wcT, bias):
    def body(p_ref, d_ref, wl_ref, wc_ref, b_ref, g_ref, d2_ref):
        hb = jnp.maximum(p_ref[0] + p_ref[1] + d_ref[...], 0.0)
        g_ref[...] = jnp.dot(hb, wl_ref[...], preferred_element_type=jnp.float32)
        d2_ref[...] = jnp.dot(hb, wc_ref[...], preferred_element_type=jnp.float32) + b_ref[...]

    return pl.pallas_call(
        body,
        grid=(N // ROW_BLK,),
        in_specs=[
            pl.BlockSpec((NC, ROW_BLK, D), lambda i: (0, i, 0)),
            pl.BlockSpec((ROW_BLK, D), lambda i: (i, 0)),
            pl.BlockSpec((D, D), lambda i: (0, 0)),
            pl.BlockSpec((D, D), lambda i: (0, 0)),
            pl.BlockSpec((1, D), lambda i: (0, 0)),
        ],
        out_specs=[
            pl.BlockSpec((ROW_BLK, D), lambda i: (i, 0)),
            pl.BlockSpec((ROW_BLK, D), lambda i: (i, 0)),
        ],
        out_shape=[
            jax.ShapeDtypeStruct((N, D), jnp.float32),
            jax.ShapeDtypeStruct((N, D), jnp.float32),
        ],
    )(p, d, wlT, wcT, bias)


def _tc_tail(p, d, owT, ob):
    def body(p_ref, d_ref, ow_ref, ob_ref, o_ref):
        hb = jnp.maximum(p_ref[0] + p_ref[1] + d_ref[...], 0.0)
        o_ref[...] = jnp.dot(hb, ow_ref[...], preferred_element_type=jnp.float32) + ob_ref[...]

    return pl.pallas_call(
        body,
        grid=(N // ROW_BLK,),
        in_specs=[
            pl.BlockSpec((NC, ROW_BLK, D), lambda i: (0, i, 0)),
            pl.BlockSpec((ROW_BLK, D), lambda i: (i, 0)),
            pl.BlockSpec((D, D), lambda i: (0, 0)),
            pl.BlockSpec((1, D), lambda i: (0, 0)),
        ],
        out_specs=pl.BlockSpec((ROW_BLK, D), lambda i: (i, 0)),
        out_shape=jax.ShapeDtypeStruct((N, D), jnp.float32),
    )(p, d, owT, ob)


def _pad_edges(ei):
    src = jnp.concatenate([ei[1], jnp.zeros((E_PAD - E,), jnp.int32)])
    dst = jnp.concatenate([ei[0], jnp.full((E_PAD - E,), N, jnp.int32)])
    return src, dst


def kernel(x, edge_index_r0, edge_index_r1,
           l0_w0_w, l0_w0_b, l0_wl_w, l0_wl_b, l0_w1_w, l0_w1_b,
           l1_w0_w, l1_w0_b, l1_wl_w, l1_wl_b, l1_w1_w, l1_w1_b,
           out_w, out_b):
    wl1T = l1_wl_w.T
    wc1T = (l1_w0_w + l1_w1_w).T
    b1 = (l1_wl_b + l1_w0_b + l1_w1_b).reshape(1, D)
    wl0T = l0_wl_w.T
    wc0T = (l0_w0_w + l0_w1_w).T
    b0 = (l0_wl_b + l0_w0_b + l0_w1_b).reshape(1, D)
    owT = out_w.T
    ob = out_b.reshape(1, D)

    src1, dst1 = _pad_edges(edge_index_r1)
    src0, dst0 = _pad_edges(edge_index_r0)
    zeros = jnp.zeros((ROWS_PER_TILE_INIT, D), jnp.float32)

    g1, d1 = _tc_head(x, wl1T, wc1T, b1)
    p1 = _sc_segment_sum(g1, src1, dst1, zeros)
    g2, d2 = _tc_mid(p1, d1, wl0T, wc0T, b0)
    p2 = _sc_segment_sum(g2, src0, dst0, zeros)
    return _tc_tail(p2, d2, owT, ob)


# speedup vs baseline: 2.7368x; 2.7368x over previous
import functools
import jax
import jax.numpy as jnp
from jax import lax
from jax.experimental import pallas as pl
from jax.experimental.pallas import tpu as pltpu
from jax.experimental.pallas import tpu_sc as plsc

N = 10000
D = 128
E = 320000

NC = 2
NS = 16
NW = NC * NS
CHUNK = 128
E_PAD = 327680
EPW = E_PAD // NW
N_CHUNKS = EPW // CHUNK
N_PAD = 10240
ROWS_PER_TILE = N_PAD // NS

_sc_mesh = plsc.VectorSubcoreMesh(core_axis_name="c", subcore_axis_name="s")


@functools.partial(
    pl.kernel,
    out_type=jax.ShapeDtypeStruct((NC, N_PAD, D), jnp.float32),
    mesh=_sc_mesh,
    scratch_types=[
        pltpu.VMEM((CHUNK,), jnp.int32),
        pltpu.VMEM((CHUNK,), jnp.int32),
        pltpu.VMEM((CHUNK, D), jnp.float32),
        pltpu.VMEM_SHARED((N_PAD, D), jnp.float32),
        pltpu.SemaphoreType.DMA,
    ],
)
def _sc_segment_sum(g_hbm, src_hbm, dst_hbm, zeros_hbm, out_hbm,
                    src_v, dst_v, rows_v, acc, sem):
    c = lax.axis_index("c")
    s = lax.axis_index("s")
    wid = s * NC + c

    pltpu.sync_copy(zeros_hbm, acc.at[pl.ds(s * ROWS_PER_TILE, ROWS_PER_TILE)])
    plsc.subcore_barrier()

    base = wid * EPW

    @pl.loop(0, N_CHUNKS)
    def _(j):
        off = base + j * CHUNK
        pltpu.sync_copy(src_hbm.at[pl.ds(off, CHUNK)], src_v)
        pltpu.sync_copy(dst_hbm.at[pl.ds(off, CHUNK)], dst_v)
        pltpu.async_copy(g_hbm.at[src_v], rows_v, sem).wait()
        pltpu.sync_copy(rows_v, acc.at[dst_v], add=True)

    plsc.subcore_barrier()

    pltpu.sync_copy(acc.at[pl.ds(s * ROWS_PER_TILE, ROWS_PER_TILE)],
                    out_hbm.at[c, pl.ds(s * ROWS_PER_TILE, ROWS_PER_TILE)])


ROW_BLK = 1000


def _tc_head(h, wlT, wcT, bias):
    def body(h_ref, wl_ref, wc_ref, b_ref, g_ref, d_ref):
        hb = h_ref[...]
        g_ref[...] = jnp.dot(hb, wl_ref[...], preferred_element_type=jnp.float32)
        d_ref[...] = jnp.dot(hb, wc_ref[...], preferred_element_type=jnp.float32) + b_ref[...]

    return pl.pallas_call(
        body,
        grid=(N // ROW_BLK,),
        in_specs=[
            pl.BlockSpec((ROW_BLK, D), lambda i: (i, 0)),
            pl.BlockSpec((D, D), lambda i: (0, 0)),
            pl.BlockSpec((D, D), lambda i: (0, 0)),
            pl.BlockSpec((1, D), lambda i: (0, 0)),
        ],
        out_specs=[
            pl.BlockSpec((ROW_BLK, D), lambda i: (i, 0)),
            pl.BlockSpec((ROW_BLK, D), lambda i: (i, 0)),
        ],
        out_shape=[
            jax.ShapeDtypeStruct((N, D), jnp.float32),
            jax.ShapeDtypeStruct((N, D), jnp.float32),
        ],
    )(h, wlT, wcT, bias)


def _tc_mid(p, d, wlT, wcT, bias):
    def body(p_ref, d_ref, wl_ref, wc_ref, b_ref, g_ref, d2_ref):
        hb = jnp.maximum(p_ref[0] + p_ref[1] + d_ref[...], 0.0)
        g_ref[...] = jnp.dot(hb, wl_ref[...], preferred_element_type=jnp.float32)
        d2_ref[...] = jnp.dot(hb, wc_ref[...], preferred_element_type=jnp.float32) + b_ref[...]

    return pl.pallas_call(
        body,
        grid=(N // ROW_BLK,),
        in_specs=[
            pl.BlockSpec((NC, ROW_BLK, D), lambda i: (0, i, 0)),
            pl.BlockSpec((ROW_BLK, D), lambda i: (i, 0)),
            pl.BlockSpec((D, D), lambda i: (0, 0)),
            pl.BlockSpec((D, D), lambda i: (0, 0)),
            pl.BlockSpec((1, D), lambda i: (0, 0)),
        ],
        out_specs=[
            pl.BlockSpec((ROW_BLK, D), lambda i: (i, 0)),
            pl.BlockSpec((ROW_BLK, D), lambda i: (i, 0)),
        ],
        out_shape=[
            jax.ShapeDtypeStruct((N, D), jnp.float32),
            jax.ShapeDtypeStruct((N, D), jnp.float32),
        ],
    )(p, d, wlT, wcT, bias)


def _tc_tail(p, d, owT, ob):
    def body(p_ref, d_ref, ow_ref, ob_ref, o_ref):
        hb = jnp.maximum(p_ref[0] + p_ref[1] + d_ref[...], 0.0)
        o_ref[...] = jnp.dot(hb, ow_ref[...], preferred_element_type=jnp.float32) + ob_ref[...]

    return pl.pallas_call(
        body,
        grid=(N // ROW_BLK,),
        in_specs=[
            pl.BlockSpec((NC, ROW_BLK, D), lambda i: (0, i, 0)),
            pl.BlockSpec((ROW_BLK, D), lambda i: (i, 0)),
            pl.BlockSpec((D, D), lambda i: (0, 0)),
            pl.BlockSpec((1, D), lambda i: (0, 0)),
        ],
        out_specs=pl.BlockSpec((ROW_BLK, D), lambda i: (i, 0)),
        out_shape=jax.ShapeDtypeStruct((N, D), jnp.float32),
    )(p, d, owT, ob)


def _pad_edges(ei):
    src = jnp.concatenate([ei[1], jnp.zeros((E_PAD - E,), jnp.int32)])
    dst = jnp.concatenate([ei[0], jnp.full((E_PAD - E,), N, jnp.int32)])
    return src, dst


def kernel(x, edge_index_r0, edge_index_r1,
           l0_w0_w, l0_w0_b, l0_wl_w, l0_wl_b, l0_w1_w, l0_w1_b,
           l1_w0_w, l1_w0_b, l1_wl_w, l1_wl_b, l1_w1_w, l1_w1_b,
           out_w, out_b):
    wl1T = l1_wl_w.T
    wc1T = (l1_w0_w + l1_w1_w).T
    b1 = (l1_wl_b + l1_w0_b + l1_w1_b).reshape(1, D)
    wl0T = l0_wl_w.T
    wc0T = (l0_w0_w + l0_w1_w).T
    b0 = (l0_wl_b + l0_w0_b + l0_w1_b).reshape(1, D)
    owT = out_w.T
    ob = out_b.reshape(1, D)

    src1, dst1 = _pad_edges(edge_index_r1)
    src0, dst0 = _pad_edges(edge_index_r0)
    zeros = jnp.zeros((ROWS_PER_TILE, D), jnp.float32)

    g1, d1 = _tc_head(x, wl1T, wc1T, b1)
    p1 = _sc_segment_sum(g1, src1, dst1, zeros)
    g2, d2 = _tc_mid(p1, d1, wl0T, wc0T, b0)
    p2 = _sc_segment_sum(g2, src0, dst0, zeros)
    return _tc_tail(p2, d2, owT, ob)


# speedup vs baseline: 3.3395x; 1.2202x over previous
import functools
import jax
import jax.numpy as jnp
from jax import lax
from jax.experimental import pallas as pl
from jax.experimental.pallas import tpu as pltpu
from jax.experimental.pallas import tpu_sc as plsc

N = 10000
D = 128
E = 320000

NC = 2
NS = 16
NW = NC * NS
CHUNK = 128
E_PAD = 327680
EPW = E_PAD // NW
N_CHUNKS = EPW // CHUNK
N_PAD = 10240
ROWS_PER_TILE = N_PAD // NS

_sc_mesh = plsc.VectorSubcoreMesh(core_axis_name="c", subcore_axis_name="s")

NBUF = 2
GROUPS = N_CHUNKS // NBUF


@functools.partial(
    pl.kernel,
    out_type=jax.ShapeDtypeStruct((NC, N_PAD, D), jnp.float32),
    mesh=_sc_mesh,
    scratch_types=[
        pltpu.VMEM((CHUNK,), jnp.int32),
        pltpu.VMEM((CHUNK,), jnp.int32),
        pltpu.VMEM((CHUNK,), jnp.int32),
        pltpu.VMEM((CHUNK,), jnp.int32),
        pltpu.VMEM((CHUNK, D), jnp.float32),
        pltpu.VMEM((CHUNK, D), jnp.float32),
        pltpu.VMEM_SHARED((N_PAD, D), jnp.float32),
        pltpu.SemaphoreType.DMA,
        pltpu.SemaphoreType.DMA,
        pltpu.SemaphoreType.DMA,
        pltpu.SemaphoreType.DMA,
        pltpu.SemaphoreType.DMA,
        pltpu.SemaphoreType.DMA,
        pltpu.SemaphoreType.DMA,
        pltpu.SemaphoreType.DMA,
    ],
)
def _sc_segment_sum(g_hbm, src_hbm, dst_hbm, zeros_hbm, out_hbm,
                    sv0, sv1, dv0, dv1, r0, r1, acc,
                    is0, is1, id0, id1, g0, g1, s0, s1):
    srcb = [sv0, sv1]
    dstb = [dv0, dv1]
    rows = [r0, r1]
    isem = [is0, is1]
    dsem = [id0, id1]
    gsem = [g0, g1]
    ssem = [s0, s1]
    c = lax.axis_index("c")
    s = lax.axis_index("s")
    wid = s * NC + c

    pltpu.sync_copy(zeros_hbm, acc.at[pl.ds(s * ROWS_PER_TILE, ROWS_PER_TILE)])

    base = wid * EPW

    for b in range(NBUF):
        off = base + b * CHUNK
        pltpu.async_copy(src_hbm.at[pl.ds(off, CHUNK)], srcb[b], isem[b])
        pltpu.async_copy(dst_hbm.at[pl.ds(off, CHUNK)], dstb[b], dsem[b])
    plsc.subcore_barrier()
    for b in range(NBUF):
        off = base + b * CHUNK
        pltpu.make_async_copy(src_hbm.at[pl.ds(off, CHUNK)], srcb[b], isem[b]).wait()
        pltpu.async_copy(g_hbm.at[srcb[b]], rows[b], gsem[b])

    @pl.loop(0, GROUPS)
    def _(grp):
        base_ch = grp * NBUF
        for b in range(NBUF):
            ch = base_ch + b
            pltpu.make_async_copy(g_hbm.at[srcb[b]], rows[b], gsem[b]).wait()
            pltpu.make_async_copy(dst_hbm.at[pl.ds(base, CHUNK)], dstb[b],
                                  dsem[b]).wait()
            pltpu.async_copy(rows[b], acc.at[dstb[b]], ssem[b], add=True)
            nxt = ch + NBUF

            @pl.when(nxt < N_CHUNKS)
            def _pf_src(b=b, nxt=nxt):
                pltpu.async_copy(src_hbm.at[pl.ds(base + nxt * CHUNK, CHUNK)],
                                 srcb[b], isem[b])
        for b in range(NBUF):
            ch = base_ch + b
            pltpu.make_async_copy(rows[b], acc.at[dstb[b]], ssem[b]).wait()
            nxt = ch + NBUF

            @pl.when(nxt < N_CHUNKS)
            def _refill(b=b, nxt=nxt):
                pltpu.async_copy(dst_hbm.at[pl.ds(base + nxt * CHUNK, CHUNK)],
                                 dstb[b], dsem[b])
                pltpu.make_async_copy(src_hbm.at[pl.ds(base, CHUNK)], srcb[b],
                                      isem[b]).wait()
                pltpu.async_copy(g_hbm.at[srcb[b]], rows[b], gsem[b])

    plsc.subcore_barrier()

    pltpu.sync_copy(acc.at[pl.ds(s * ROWS_PER_TILE, ROWS_PER_TILE)],
                    out_hbm.at[c, pl.ds(s * ROWS_PER_TILE, ROWS_PER_TILE)])


ROW_BLK = 1000


def _tc_head(h, wlT, wcT, bias):
    def body(h_ref, wl_ref, wc_ref, b_ref, g_ref, d_ref):
        hb = h_ref[...]
        g_ref[...] = jnp.dot(hb, wl_ref[...], preferred_element_type=jnp.float32)
        d_ref[...] = jnp.dot(hb, wc_ref[...], preferred_element_type=jnp.float32) + b_ref[...]

    return pl.pallas_call(
        body,
        grid=(N // ROW_BLK,),
        in_specs=[
            pl.BlockSpec((ROW_BLK, D), lambda i: (i, 0)),
            pl.BlockSpec((D, D), lambda i: (0, 0)),
            pl.BlockSpec((D, D), lambda i: (0, 0)),
            pl.BlockSpec((1, D), lambda i: (0, 0)),
        ],
        out_specs=[
            pl.BlockSpec((ROW_BLK, D), lambda i: (i, 0)),
            pl.BlockSpec((ROW_BLK, D), lambda i: (i, 0)),
        ],
        out_shape=[
            jax.ShapeDtypeStruct((N, D), jnp.float32),
            jax.ShapeDtypeStruct((N, D), jnp.float32),
        ],
    )(h, wlT, wcT, bias)


def _tc_mid(p, d, wlT, wcT, bias):
    def body(p_ref, d_ref, wl_ref, wc_ref, b_ref, g_ref, d2_ref):
        hb = jnp.maximum(p_ref[0] + p_ref[1] + d_ref[...], 0.0)
        g_ref[...] = jnp.dot(hb, wl_ref[...], preferred_element_type=jnp.float32)
        d2_ref[...] = jnp.dot(hb, wc_ref[...], preferred_element_type=jnp.float32) + b_ref[...]

    return pl.pallas_call(
        body,
        grid=(N // ROW_BLK,),
        in_specs=[
            pl.BlockSpec((NC, ROW_BLK, D), lambda i: (0, i, 0)),
            pl.BlockSpec((ROW_BLK, D), lambda i: (i, 0)),
            pl.BlockSpec((D, D), lambda i: (0, 0)),
            pl.BlockSpec((D, D), lambda i: (0, 0)),
            pl.BlockSpec((1, D), lambda i: (0, 0)),
        ],
        out_specs=[
            pl.BlockSpec((ROW_BLK, D), lambda i: (i, 0)),
            pl.BlockSpec((ROW_BLK, D), lambda i: (i, 0)),
        ],
        out_shape=[
            jax.ShapeDtypeStruct((N, D), jnp.float32),
            jax.ShapeDtypeStruct((N, D), jnp.float32),
        ],
    )(p, d, wlT, wcT, bias)


def _tc_tail(p, d, owT, ob):
    def body(p_ref, d_ref, ow_ref, ob_ref, o_ref):
        hb = jnp.maximum(p_ref[0] + p_ref[1] + d_ref[...], 0.0)
        o_ref[...] = jnp.dot(hb, ow_ref[...], preferred_element_type=jnp.float32) + ob_ref[...]

    return pl.pallas_call(
        body,
        grid=(N // ROW_BLK,),
        in_specs=[
            pl.BlockSpec((NC, ROW_BLK, D), lambda i: (0, i, 0)),
            pl.BlockSpec((ROW_BLK, D), lambda i: (i, 0)),
            pl.BlockSpec((D, D), lambda i: (0, 0)),
            pl.BlockSpec((1, D), lambda i: (0, 0)),
        ],
        out_specs=pl.BlockSpec((ROW_BLK, D), lambda i: (i, 0)),
        out_shape=jax.ShapeDtypeStruct((N, D), jnp.float32),
    )(p, d, owT, ob)


def _pad_edges(ei):
    src = jnp.concatenate([ei[1], jnp.zeros((E_PAD - E,), jnp.int32)])
    dst = jnp.concatenate([ei[0], jnp.full((E_PAD - E,), N, jnp.int32)])
    return src, dst


def kernel(x, edge_index_r0, edge_index_r1,
           l0_w0_w, l0_w0_b, l0_wl_w, l0_wl_b, l0_w1_w, l0_w1_b,
           l1_w0_w, l1_w0_b, l1_wl_w, l1_wl_b, l1_w1_w, l1_w1_b,
           out_w, out_b):
    wl1T = l1_wl_w.T
    wc1T = (l1_w0_w + l1_w1_w).T
    b1 = (l1_wl_b + l1_w0_b + l1_w1_b).reshape(1, D)
    wl0T = l0_wl_w.T
    wc0T = (l0_w0_w + l0_w1_w).T
    b0 = (l0_wl_b + l0_w0_b + l0_w1_b).reshape(1, D)
    owT = out_w.T
    ob = out_b.reshape(1, D)

    src1, dst1 = _pad_edges(edge_index_r1)
    src0, dst0 = _pad_edges(edge_index_r0)
    zeros = jnp.zeros((ROWS_PER_TILE, D), jnp.float32)

    g1, d1 = _tc_head(x, wl1T, wc1T, b1)
    p1 = _sc_segment_sum(g1, src1, dst1, zeros)
    g2, d2 = _tc_mid(p1, d1, wl0T, wc0T, b0)
    p2 = _sc_segment_sum(g2, src0, dst0, zeros)
    return _tc_tail(p2, d2, owT, ob)


# speedup vs baseline: 9.1285x; 2.7335x over previous
import functools
import jax
import jax.numpy as jnp
from jax import lax
from jax.experimental import pallas as pl
from jax.experimental.pallas import tpu as pltpu
from jax.experimental.pallas import tpu_sc as plsc

N = 10000
D = 128
E = 320000

NC = 2
NS = 16
NW = NC * NS
CHUNK = 128
E_PAD = 327680
EPW = E_PAD // NW
N_CHUNKS = EPW // CHUNK
N_PAD = 10240
ROWS_PER_TILE = N_PAD // NS

_sc_mesh = plsc.VectorSubcoreMesh(core_axis_name="c", subcore_axis_name="s")

NBUF = 2
GROUPS = N_CHUNKS // NBUF


@functools.partial(
    pl.kernel,
    out_type=jax.ShapeDtypeStruct((NC, N_PAD, D), jnp.float32),
    mesh=_sc_mesh,
    scratch_types=[
        pltpu.VMEM((CHUNK,), jnp.int32),
        pltpu.VMEM((CHUNK,), jnp.int32),
        pltpu.VMEM((CHUNK,), jnp.int32),
        pltpu.VMEM((CHUNK,), jnp.int32),
        pltpu.VMEM((CHUNK, D), jnp.float32),
        pltpu.VMEM((CHUNK, D), jnp.float32),
        pltpu.VMEM_SHARED((N_PAD, D), jnp.float32),
        pltpu.SemaphoreType.DMA,
        pltpu.SemaphoreType.DMA,
        pltpu.SemaphoreType.DMA,
        pltpu.SemaphoreType.DMA,
        pltpu.SemaphoreType.DMA,
        pltpu.SemaphoreType.DMA,
        pltpu.SemaphoreType.DMA,
        pltpu.SemaphoreType.DMA,
    ],
)
def _sc_segment_sum(g_hbm, src_hbm, dst_hbm, zeros_hbm, out_hbm,
                    sv0, sv1, dv0, dv1, r0, r1, acc,
                    is0, is1, id0, id1, g0, g1, s0, s1):
    srcb = [sv0, sv1]
    dstb = [dv0, dv1]
    rows = [r0, r1]
    isem = [is0, is1]
    dsem = [id0, id1]
    gsem = [g0, g1]
    ssem = [s0, s1]
    c = lax.axis_index("c")
    s = lax.axis_index("s")
    wid = s * NC + c

    pltpu.sync_copy(zeros_hbm, acc.at[pl.ds(s * ROWS_PER_TILE, ROWS_PER_TILE)])

    base = wid * EPW

    for b in range(NBUF):
        off = base + b * CHUNK
        pltpu.async_copy(src_hbm.at[pl.ds(off, CHUNK)], srcb[b], isem[b])
        pltpu.async_copy(dst_hbm.at[pl.ds(off, CHUNK)], dstb[b], dsem[b])
    plsc.subcore_barrier()
    for b in range(NBUF):
        off = base + b * CHUNK
        pltpu.make_async_copy(src_hbm.at[pl.ds(off, CHUNK)], srcb[b], isem[b]).wait()
        pltpu.async_copy(g_hbm.at[srcb[b]], rows[b], gsem[b])

    @pl.loop(0, GROUPS)
    def _(grp):
        base_ch = grp * NBUF
        for b in range(NBUF):
            ch = base_ch + b
            pltpu.make_async_copy(g_hbm.at[srcb[b]], rows[b], gsem[b]).wait()
            pltpu.make_async_copy(dst_hbm.at[pl.ds(base, CHUNK)], dstb[b],
                                  dsem[b]).wait()
            pltpu.async_copy(rows[b], acc.at[dstb[b]], ssem[b], add=True)
            nxt = ch + NBUF

            @pl.when(nxt < N_CHUNKS)
            def _pf_src(b=b, nxt=nxt):
                pltpu.async_copy(src_hbm.at[pl.ds(base + nxt * CHUNK, CHUNK)],
                                 srcb[b], isem[b])
        for b in range(NBUF):
            ch = base_ch + b
            pltpu.make_async_copy(rows[b], acc.at[dstb[b]], ssem[b]).wait()
            nxt = ch + NBUF

            @pl.when(nxt < N_CHUNKS)
            def _refill(b=b, nxt=nxt):
                pltpu.async_copy(dst_hbm.at[pl.ds(base + nxt * CHUNK, CHUNK)],
                                 dstb[b], dsem[b])
                pltpu.make_async_copy(src_hbm.at[pl.ds(base, CHUNK)], srcb[b],
                                      isem[b]).wait()
                pltpu.async_copy(g_hbm.at[srcb[b]], rows[b], gsem[b])

    plsc.subcore_barrier()

    pltpu.sync_copy(acc.at[pl.ds(s * ROWS_PER_TILE, ROWS_PER_TILE)],
                    out_hbm.at[c, pl.ds(s * ROWS_PER_TILE, ROWS_PER_TILE)])


ROW_BLK = 1000


def _tc_head(h, wlT, wcT, bias):
    def body(h_ref, wl_ref, wc_ref, b_ref, g_ref, d_ref):
        hb = h_ref[...]
        g_ref[...] = jnp.dot(hb, wl_ref[...], preferred_element_type=jnp.float32)
        d_ref[...] = jnp.dot(hb, wc_ref[...], preferred_element_type=jnp.float32) + b_ref[...]

    return pl.pallas_call(
        body,
        grid=(N // ROW_BLK,),
        in_specs=[
            pl.BlockSpec((ROW_BLK, D), lambda i: (i, 0)),
            pl.BlockSpec((D, D), lambda i: (0, 0)),
            pl.BlockSpec((D, D), lambda i: (0, 0)),
            pl.BlockSpec((1, D), lambda i: (0, 0)),
        ],
        out_specs=[
            pl.BlockSpec((ROW_BLK, D), lambda i: (i, 0)),
            pl.BlockSpec((ROW_BLK, D), lambda i: (i, 0)),
        ],
        out_shape=[
            jax.ShapeDtypeStruct((N, D), jnp.float32),
            jax.ShapeDtypeStruct((N, D), jnp.float32),
        ],
    )(h, wlT, wcT, bias)


def _tc_mid(p, d, wlT, wcT, bias):
    def body(p_ref, d_ref, wl_ref, wc_ref, b_ref, g_ref, d2_ref):
        hb = jnp.maximum(p_ref[0] + p_ref[1] + d_ref[...], 0.0)
        g_ref[...] = jnp.dot(hb, wl_ref[...], preferred_element_type=jnp.float32)
        d2_ref[...] = jnp.dot(hb, wc_ref[...], preferred_element_type=jnp.float32) + b_ref[...]

    return pl.pallas_call(
        body,
        grid=(N // ROW_BLK,),
        in_specs=[
            pl.BlockSpec((NC, ROW_BLK, D), lambda i: (0, i, 0)),
            pl.BlockSpec((ROW_BLK, D), lambda i: (i, 0)),
            pl.BlockSpec((D, D), lambda i: (0, 0)),
            pl.BlockSpec((D, D), lambda i: (0, 0)),
            pl.BlockSpec((1, D), lambda i: (0, 0)),
        ],
        out_specs=[
            pl.BlockSpec((ROW_BLK, D), lambda i: (i, 0)),
            pl.BlockSpec((ROW_BLK, D), lambda i: (i, 0)),
        ],
        out_shape=[
            jax.ShapeDtypeStruct((N, D), jnp.float32),
            jax.ShapeDtypeStruct((N, D), jnp.float32),
        ],
    )(p, d, wlT, wcT, bias)


def _tc_tail(p, d, owT, ob):
    def body(p_ref, d_ref, ow_ref, ob_ref, o_ref):
        hb = jnp.maximum(p_ref[0] + p_ref[1] + d_ref[...], 0.0)
        o_ref[...] = jnp.dot(hb, ow_ref[...], preferred_element_type=jnp.float32) + ob_ref[...]

    return pl.pallas_call(
        body,
        grid=(N // ROW_BLK,),
        in_specs=[
            pl.BlockSpec((NC, ROW_BLK, D), lambda i: (0, i, 0)),
            pl.BlockSpec((ROW_BLK, D), lambda i: (i, 0)),
            pl.BlockSpec((D, D), lambda i: (0, 0)),
            pl.BlockSpec((1, D), lambda i: (0, 0)),
        ],
        out_specs=pl.BlockSpec((ROW_BLK, D), lambda i: (i, 0)),
        out_shape=jax.ShapeDtypeStruct((N, D), jnp.float32),
    )(p, d, owT, ob)


def _pad_edges(ei):
    pad = jnp.arange(E_PAD - E, dtype=jnp.int32)
    src = jnp.concatenate([ei[1], pad % N])
    dst = jnp.concatenate([ei[0], N + pad % (N_PAD - N)])
    return src, dst


def kernel(x, edge_index_r0, edge_index_r1,
           l0_w0_w, l0_w0_b, l0_wl_w, l0_wl_b, l0_w1_w, l0_w1_b,
           l1_w0_w, l1_w0_b, l1_wl_w, l1_wl_b, l1_w1_w, l1_w1_b,
           out_w, out_b):
    wl1T = l1_wl_w.T
    wc1T = (l1_w0_w + l1_w1_w).T
    b1 = (l1_wl_b + l1_w0_b + l1_w1_b).reshape(1, D)
    wl0T = l0_wl_w.T
    wc0T = (l0_w0_w + l0_w1_w).T
    b0 = (l0_wl_b + l0_w0_b + l0_w1_b).reshape(1, D)
    owT = out_w.T
    ob = out_b.reshape(1, D)

    src1, dst1 = _pad_edges(edge_index_r1)
    src0, dst0 = _pad_edges(edge_index_r0)
    zeros = jnp.zeros((ROWS_PER_TILE, D), jnp.float32)

    g1, d1 = _tc_head(x, wl1T, wc1T, b1)
    p1 = _sc_segment_sum(g1, src1, dst1, zeros)
    g2, d2 = _tc_mid(p1, d1, wl0T, wc0T, b0)
    p2 = _sc_segment_sum(g2, src0, dst0, zeros)
    return _tc_tail(p2, d2, owT, ob)


# speedup vs baseline: 10.8209x; 1.1854x over previous
import functools
import jax
import jax.numpy as jnp
from jax import lax
from jax.experimental import pallas as pl
from jax.experimental.pallas import tpu as pltpu
from jax.experimental.pallas import tpu_sc as plsc

N = 10000
D = 128
E = 320000

NC = 2
NS = 16
NW = NC * NS
CHUNK = 64
E_PAD = 327680
EPW = E_PAD // NW
N_CHUNKS = EPW // CHUNK
N_PAD = 10240
ROWS_PER_TILE = N_PAD // NS

_sc_mesh = plsc.VectorSubcoreMesh(core_axis_name="c", subcore_axis_name="s")

NBUF = 4
GROUPS = N_CHUNKS // NBUF

_scratch = []
for _ in range(NBUF):
    _scratch.append(pltpu.VMEM((CHUNK,), jnp.int32))
    _scratch.append(pltpu.VMEM((CHUNK,), jnp.int32))
    _scratch.append(pltpu.VMEM((CHUNK, D), jnp.float32))
_scratch.append(pltpu.VMEM_SHARED((N_PAD, D), jnp.float32))
_scratch.extend([pltpu.SemaphoreType.DMA] * (4 * NBUF))


@functools.partial(
    pl.kernel,
    out_type=jax.ShapeDtypeStruct((NC, N_PAD, D), jnp.float32),
    mesh=_sc_mesh,
    scratch_types=_scratch,
)
def _sc_segment_sum(g_hbm, src_hbm, dst_hbm, zeros_hbm, out_hbm, *scr):
    srcb = [scr[3 * b] for b in range(NBUF)]
    dstb = [scr[3 * b + 1] for b in range(NBUF)]
    rows = [scr[3 * b + 2] for b in range(NBUF)]
    acc = scr[3 * NBUF]
    sems = scr[3 * NBUF + 1:]
    isem = sems[0:NBUF]
    dsem = sems[NBUF:2 * NBUF]
    gsem = sems[2 * NBUF:3 * NBUF]
    ssem = sems[3 * NBUF:4 * NBUF]
    c = lax.axis_index("c")
    s = lax.axis_index("s")
    wid = s * NC + c

    pltpu.sync_copy(zeros_hbm, acc.at[pl.ds(s * ROWS_PER_TILE, ROWS_PER_TILE)])

    base = wid * EPW

    for b in range(NBUF):
        off = base + b * CHUNK
        pltpu.async_copy(src_hbm.at[pl.ds(off, CHUNK)], srcb[b], isem[b])
        pltpu.async_copy(dst_hbm.at[pl.ds(off, CHUNK)], dstb[b], dsem[b])
    plsc.subcore_barrier()
    for b in range(NBUF):
        off = base + b * CHUNK
        pltpu.make_async_copy(src_hbm.at[pl.ds(off, CHUNK)], srcb[b], isem[b]).wait()
        pltpu.async_copy(g_hbm.at[srcb[b]], rows[b], gsem[b])

    @pl.loop(0, GROUPS)
    def _(grp):
        base_ch = grp * NBUF
        for b in range(NBUF):
            ch = base_ch + b
            pltpu.make_async_copy(g_hbm.at[srcb[b]], rows[b], gsem[b]).wait()
            pltpu.make_async_copy(dst_hbm.at[pl.ds(base, CHUNK)], dstb[b],
                                  dsem[b]).wait()
            pltpu.async_copy(rows[b], acc.at[dstb[b]], ssem[b], add=True)
            nxt = ch + NBUF

            @pl.when(nxt < N_CHUNKS)
            def _pf_src(b=b, nxt=nxt):
                pltpu.async_copy(src_hbm.at[pl.ds(base + nxt * CHUNK, CHUNK)],
                                 srcb[b], isem[b])
        for b in range(NBUF):
            ch = base_ch + b
            pltpu.make_async_copy(rows[b], acc.at[dstb[b]], ssem[b]).wait()
            nxt = ch + NBUF

            @pl.when(nxt < N_CHUNKS)
            def _refill(b=b, nxt=nxt):
                pltpu.async_copy(dst_hbm.at[pl.ds(base + nxt * CHUNK, CHUNK)],
                                 dstb[b], dsem[b])
                pltpu.make_async_copy(src_hbm.at[pl.ds(base, CHUNK)], srcb[b],
                                      isem[b]).wait()
                pltpu.async_copy(g_hbm.at[srcb[b]], rows[b], gsem[b])

    plsc.subcore_barrier()

    pltpu.sync_copy(acc.at[pl.ds(s * ROWS_PER_TILE, ROWS_PER_TILE)],
                    out_hbm.at[c, pl.ds(s * ROWS_PER_TILE, ROWS_PER_TILE)])


ROW_BLK = 1000


def _tc_head(h, wlT, wcT, bias):
    def body(h_ref, wl_ref, wc_ref, b_ref, g_ref, d_ref):
        hb = h_ref[...]
        g_ref[...] = jnp.dot(hb, wl_ref[...], preferred_element_type=jnp.float32)
        d_ref[...] = jnp.dot(hb, wc_ref[...], preferred_element_type=jnp.float32) + b_ref[...]

    return pl.pallas_call(
        body,
        grid=(N // ROW_BLK,),
        in_specs=[
            pl.BlockSpec((ROW_BLK, D), lambda i: (i, 0)),
            pl.BlockSpec((D, D), lambda i: (0, 0)),
            pl.BlockSpec((D, D), lambda i: (0, 0)),
            pl.BlockSpec((1, D), lambda i: (0, 0)),
        ],
        out_specs=[
            pl.BlockSpec((ROW_BLK, D), lambda i: (i, 0)),
            pl.BlockSpec((ROW_BLK, D), lambda i: (i, 0)),
        ],
        out_shape=[
            jax.ShapeDtypeStruct((N, D), jnp.float32),
            jax.ShapeDtypeStruct((N, D), jnp.float32),
        ],
    )(h, wlT, wcT, bias)


def _tc_mid(p, d, wlT, wcT, bias):
    def body(p_ref, d_ref, wl_ref, wc_ref, b_ref, g_ref, d2_ref):
        hb = jnp.maximum(p_ref[0] + p_ref[1] + d_ref[...], 0.0)
        g_ref[...] = jnp.dot(hb, wl_ref[...], preferred_element_type=jnp.float32)
        d2_ref[...] = jnp.dot(hb, wc_ref[...], preferred_element_type=jnp.float32) + b_ref[...]

    return pl.pallas_call(
        body,
        grid=(N // ROW_BLK,),
        in_specs=[
            pl.BlockSpec((NC, ROW_BLK, D), lambda i: (0, i, 0)),
            pl.BlockSpec((ROW_BLK, D), lambda i: (i, 0)),
            pl.BlockSpec((D, D), lambda i: (0, 0)),
            pl.BlockSpec((D, D), lambda i: (0, 0)),
            pl.BlockSpec((1, D), lambda i: (0, 0)),
        ],
        out_specs=[
            pl.BlockSpec((ROW_BLK, D), lambda i: (i, 0)),
            pl.BlockSpec((ROW_BLK, D), lambda i: (i, 0)),
        ],
        out_shape=[
            jax.ShapeDtypeStruct((N, D), jnp.float32),
            jax.ShapeDtypeStruct((N, D), jnp.float32),
        ],
    )(p, d, wlT, wcT, bias)


def _tc_tail(p, d, owT, ob):
    def body(p_ref, d_ref, ow_ref, ob_ref, o_ref):
        hb = jnp.maximum(p_ref[0] + p_ref[1] + d_ref[...], 0.0)
        o_ref[...] = jnp.dot(hb, ow_ref[...], preferred_element_type=jnp.float32) + ob_ref[...]

    return pl.pallas_call(
        body,
        grid=(N // ROW_BLK,),
        in_specs=[
            pl.BlockSpec((NC, ROW_BLK, D), lambda i: (0, i, 0)),
            pl.BlockSpec((ROW_BLK, D), lambda i: (i, 0)),
            pl.BlockSpec((D, D), lambda i: (0, 0)),
            pl.BlockSpec((1, D), lambda i: (0, 0)),
        ],
        out_specs=pl.BlockSpec((ROW_BLK, D), lambda i: (i, 0)),
        out_shape=jax.ShapeDtypeStruct((N, D), jnp.float32),
    )(p, d, owT, ob)


def _pad_edges(ei):
    pad = jnp.arange(E_PAD - E, dtype=jnp.int32)
    src = jnp.concatenate([ei[1], pad % N])
    dst = jnp.concatenate([ei[0], N + pad % (N_PAD - N)])
    return src, dst


def kernel(x, edge_index_r0, edge_index_r1,
           l0_w0_w, l0_w0_b, l0_wl_w, l0_wl_b, l0_w1_w, l0_w1_b,
           l1_w0_w, l1_w0_b, l1_wl_w, l1_wl_b, l1_w1_w, l1_w1_b,
           out_w, out_b):
    wl1T = l1_wl_w.T
    wc1T = (l1_w0_w + l1_w1_w).T
    b1 = (l1_wl_b + l1_w0_b + l1_w1_b).reshape(1, D)
    wl0T = l0_wl_w.T
    wc0T = (l0_w0_w + l0_w1_w).T
    b0 = (l0_wl_b + l0_w0_b + l0_w1_b).reshape(1, D)
    owT = out_w.T
    ob = out_b.reshape(1, D)

    src1, dst1 = _pad_edges(edge_index_r1)
    src0, dst0 = _pad_edges(edge_index_r0)
    zeros = jnp.zeros((ROWS_PER_TILE, D), jnp.float32)

    g1, d1 = _tc_head(x, wl1T, wc1T, b1)
    p1 = _sc_segment_sum(g1, src1, dst1, zeros)
    g2, d2 = _tc_mid(p1, d1, wl0T, wc0T, b0)
    p2 = _sc_segment_sum(g2, src0, dst0, zeros)
    return _tc_tail(p2, d2, owT, ob)


# speedup vs baseline: 11.0564x; 1.0218x over previous
import functools
import jax
import jax.numpy as jnp
from jax import lax
from jax.experimental import pallas as pl
from jax.experimental.pallas import tpu as pltpu
from jax.experimental.pallas import tpu_sc as plsc

N = 10000
D = 128
E = 320000

NC = 2
NS = 16
NW = NC * NS
CHUNK = 64
E_PAD = 327680
EPW = E_PAD // NW
N_CHUNKS = EPW // CHUNK
N_PAD = 10240
ROWS_PER_TILE = N_PAD // NS

_sc_mesh = plsc.VectorSubcoreMesh(core_axis_name="c", subcore_axis_name="s")

NBUF = 5
GROUPS = N_CHUNKS // NBUF

_scratch = []
for _ in range(NBUF):
    _scratch.append(pltpu.VMEM((CHUNK,), jnp.int32))
    _scratch.append(pltpu.VMEM((CHUNK,), jnp.int32))
    _scratch.append(pltpu.VMEM((CHUNK, D), jnp.float32))
_scratch.append(pltpu.VMEM_SHARED((N_PAD, D), jnp.float32))
_scratch.extend([pltpu.SemaphoreType.DMA] * (4 * NBUF))


@functools.partial(
    pl.kernel,
    out_type=jax.ShapeDtypeStruct((NC, N_PAD, D), jnp.float32),
    mesh=_sc_mesh,
    scratch_types=_scratch,
)
def _sc_segment_sum(g_hbm, src_hbm, dst_hbm, zeros_hbm, out_hbm, *scr):
    srcb = [scr[3 * b] for b in range(NBUF)]
    dstb = [scr[3 * b + 1] for b in range(NBUF)]
    rows = [scr[3 * b + 2] for b in range(NBUF)]
    acc = scr[3 * NBUF]
    sems = scr[3 * NBUF + 1:]
    isem = sems[0:NBUF]
    dsem = sems[NBUF:2 * NBUF]
    gsem = sems[2 * NBUF:3 * NBUF]
    ssem = sems[3 * NBUF:4 * NBUF]
    c = lax.axis_index("c")
    s = lax.axis_index("s")
    wid = s * NC + c

    pltpu.sync_copy(zeros_hbm, acc.at[pl.ds(s * ROWS_PER_TILE, ROWS_PER_TILE)])

    base = wid * EPW

    for b in range(NBUF):
        off = base + b * CHUNK
        pltpu.async_copy(src_hbm.at[pl.ds(off, CHUNK)], srcb[b], isem[b])
        pltpu.async_copy(dst_hbm.at[pl.ds(off, CHUNK)], dstb[b], dsem[b])
    plsc.subcore_barrier()
    for b in range(NBUF):
        off = base + b * CHUNK
        pltpu.make_async_copy(src_hbm.at[pl.ds(off, CHUNK)], srcb[b], isem[b]).wait()
        pltpu.async_copy(g_hbm.at[srcb[b]], rows[b], gsem[b])

    @pl.loop(0, GROUPS)
    def _(grp):
        base_ch = grp * NBUF
        for b in range(NBUF):
            ch = base_ch + b
            pltpu.make_async_copy(g_hbm.at[srcb[b]], rows[b], gsem[b]).wait()
            pltpu.make_async_copy(dst_hbm.at[pl.ds(base, CHUNK)], dstb[b],
                                  dsem[b]).wait()
            pltpu.async_copy(rows[b], acc.at[dstb[b]], ssem[b], add=True)
            nxt = ch + NBUF

            @pl.when(nxt < N_CHUNKS)
            def _pf_src(b=b, nxt=nxt):
                pltpu.async_copy(src_hbm.at[pl.ds(base + nxt * CHUNK, CHUNK)],
                                 srcb[b], isem[b])
        for b in range(NBUF):
            ch = base_ch + b
            pltpu.make_async_copy(rows[b], acc.at[dstb[b]], ssem[b]).wait()
            nxt = ch + NBUF

            @pl.when(nxt < N_CHUNKS)
            def _refill(b=b, nxt=nxt):
                pltpu.async_copy(dst_hbm.at[pl.ds(base + nxt * CHUNK, CHUNK)],
                                 dstb[b], dsem[b])
                pltpu.make_async_copy(src_hbm.at[pl.ds(base, CHUNK)], srcb[b],
                                      isem[b]).wait()
                pltpu.async_copy(g_hbm.at[srcb[b]], rows[b], gsem[b])

    plsc.subcore_barrier()

    pltpu.sync_copy(acc.at[pl.ds(s * ROWS_PER_TILE, ROWS_PER_TILE)],
                    out_hbm.at[c, pl.ds(s * ROWS_PER_TILE, ROWS_PER_TILE)])


ROW_BLK = 1000


def _tc_head(h, wlT, wcT, bias):
    def body(h_ref, wl_ref, wc_ref, b_ref, g_ref, d_ref):
        hb = h_ref[...]
        g_ref[...] = jnp.dot(hb, wl_ref[...], preferred_element_type=jnp.float32)
        d_ref[...] = jnp.dot(hb, wc_ref[...], preferred_element_type=jnp.float32) + b_ref[...]

    return pl.pallas_call(
        body,
        grid=(N // ROW_BLK,),
        in_specs=[
            pl.BlockSpec((ROW_BLK, D), lambda i: (i, 0)),
            pl.BlockSpec((D, D), lambda i: (0, 0)),
            pl.BlockSpec((D, D), lambda i: (0, 0)),
            pl.BlockSpec((1, D), lambda i: (0, 0)),
        ],
        out_specs=[
            pl.BlockSpec((ROW_BLK, D), lambda i: (i, 0)),
            pl.BlockSpec((ROW_BLK, D), lambda i: (i, 0)),
        ],
        out_shape=[
            jax.ShapeDtypeStruct((N, D), jnp.float32),
            jax.ShapeDtypeStruct((N, D), jnp.float32),
        ],
    )(h, wlT, wcT, bias)


def _tc_mid(p, d, wlT, wcT, bias):
    def body(p_ref, d_ref, wl_ref, wc_ref, b_ref, g_ref, d2_ref):
        hb = jnp.maximum(p_ref[0] + p_ref[1] + d_ref[...], 0.0)
        g_ref[...] = jnp.dot(hb, wl_ref[...], preferred_element_type=jnp.float32)
        d2_ref[...] = jnp.dot(hb, wc_ref[...], preferred_element_type=jnp.float32) + b_ref[...]

    return pl.pallas_call(
        body,
        grid=(N // ROW_BLK,),
        in_specs=[
            pl.BlockSpec((NC, ROW_BLK, D), lambda i: (0, i, 0)),
            pl.BlockSpec((ROW_BLK, D), lambda i: (i, 0)),
            pl.BlockSpec((D, D), lambda i: (0, 0)),
            pl.BlockSpec((D, D), lambda i: (0, 0)),
            pl.BlockSpec((1, D), lambda i: (0, 0)),
        ],
        out_specs=[
            pl.BlockSpec((ROW_BLK, D), lambda i: (i, 0)),
            pl.BlockSpec((ROW_BLK, D), lambda i: (i, 0)),
        ],
        out_shape=[
            jax.ShapeDtypeStruct((N, D), jnp.float32),
            jax.ShapeDtypeStruct((N, D), jnp.float32),
        ],
    )(p, d, wlT, wcT, bias)


def _tc_tail(p, d, owT, ob):
    def body(p_ref, d_ref, ow_ref, ob_ref, o_ref):
        hb = jnp.maximum(p_ref[0] + p_ref[1] + d_ref[...], 0.0)
        o_ref[...] = jnp.dot(hb, ow_ref[...], preferred_element_type=jnp.float32) + ob_ref[...]

    return pl.pallas_call(
        body,
        grid=(N // ROW_BLK,),
        in_specs=[
            pl.BlockSpec((NC, ROW_BLK, D), lambda i: (0, i, 0)),
            pl.BlockSpec((ROW_BLK, D), lambda i: (i, 0)),
            pl.BlockSpec((D, D), lambda i: (0, 0)),
            pl.BlockSpec((1, D), lambda i: (0, 0)),
        ],
        out_specs=pl.BlockSpec((ROW_BLK, D), lambda i: (i, 0)),
        out_shape=jax.ShapeDtypeStruct((N, D), jnp.float32),
    )(p, d, owT, ob)


def _pad_edges(ei):
    pad = jnp.arange(E_PAD - E, dtype=jnp.int32)
    src = jnp.concatenate([ei[1], pad % N])
    dst = jnp.concatenate([ei[0], N + pad % (N_PAD - N)])
    return src, dst


def kernel(x, edge_index_r0, edge_index_r1,
           l0_w0_w, l0_w0_b, l0_wl_w, l0_wl_b, l0_w1_w, l0_w1_b,
           l1_w0_w, l1_w0_b, l1_wl_w, l1_wl_b, l1_w1_w, l1_w1_b,
           out_w, out_b):
    wl1T = l1_wl_w.T
    wc1T = (l1_w0_w + l1_w1_w).T
    b1 = (l1_wl_b + l1_w0_b + l1_w1_b).reshape(1, D)
    wl0T = l0_wl_w.T
    wc0T = (l0_w0_w + l0_w1_w).T
    b0 = (l0_wl_b + l0_w0_b + l0_w1_b).reshape(1, D)
    owT = out_w.T
    ob = out_b.reshape(1, D)

    src1, dst1 = _pad_edges(edge_index_r1)
    src0, dst0 = _pad_edges(edge_index_r0)
    zeros = jnp.zeros((ROWS_PER_TILE, D), jnp.float32)

    g1, d1 = _tc_head(x, wl1T, wc1T, b1)
    p1 = _sc_segment_sum(g1, src1, dst1, zeros)
    g2, d2 = _tc_mid(p1, d1, wl0T, wc0T, b0)
    p2 = _sc_segment_sum(g2, src0, dst0, zeros)
    return _tc_tail(p2, d2, owT, ob)


# speedup vs baseline: 11.3000x; 1.0220x over previous
import functools
import jax
import jax.numpy as jnp
from jax import lax
from jax.experimental import pallas as pl
from jax.experimental.pallas import tpu as pltpu
from jax.experimental.pallas import tpu_sc as plsc

N = 10000
D = 128
E = 320000

NC = 2
NS = 16
NW = NC * NS
CHUNK = 64
E_PAD = 327680
EPW = E_PAD // NW
N_CHUNKS = EPW // CHUNK
N_PAD = 10240
ROWS_PER_TILE = N_PAD // NS

_sc_mesh = plsc.VectorSubcoreMesh(core_axis_name="c", subcore_axis_name="s")

NBUF = 5
GROUPS = N_CHUNKS // NBUF

_scratch = []
for _ in range(NBUF):
    _scratch.append(pltpu.VMEM((CHUNK,), jnp.int32))
    _scratch.append(pltpu.VMEM((CHUNK,), jnp.int32))
    _scratch.append(pltpu.VMEM((CHUNK, D), jnp.float32))
_scratch.append(pltpu.VMEM_SHARED((N_PAD, D), jnp.float32))
_scratch.extend([pltpu.SemaphoreType.DMA] * (4 * NBUF))


@functools.partial(
    pl.kernel,
    out_type=jax.ShapeDtypeStruct((NC, N_PAD, D), jnp.float32),
    mesh=_sc_mesh,
    scratch_types=_scratch,
)
def _sc_segment_sum(g_hbm, src_hbm, dst_hbm, zeros_hbm, out_hbm, *scr):
    srcb = [scr[3 * b] for b in range(NBUF)]
    dstb = [scr[3 * b + 1] for b in range(NBUF)]
    rows = [scr[3 * b + 2] for b in range(NBUF)]
    acc = scr[3 * NBUF]
    sems = scr[3 * NBUF + 1:]
    isem = sems[0:NBUF]
    dsem = sems[NBUF:2 * NBUF]
    gsem = sems[2 * NBUF:3 * NBUF]
    ssem = sems[3 * NBUF:4 * NBUF]
    c = lax.axis_index("c")
    s = lax.axis_index("s")
    wid = s * NC + c

    pltpu.sync_copy(zeros_hbm, acc.at[pl.ds(s * ROWS_PER_TILE, ROWS_PER_TILE)])

    base = wid * EPW

    for b in range(NBUF):
        off = base + b * CHUNK
        pltpu.async_copy(src_hbm.at[pl.ds(off, CHUNK)], srcb[b], isem[b])
        pltpu.async_copy(dst_hbm.at[pl.ds(off, CHUNK)], dstb[b], dsem[b])
    plsc.subcore_barrier()
    for b in range(NBUF):
        off = base + b * CHUNK
        pltpu.make_async_copy(src_hbm.at[pl.ds(off, CHUNK)], srcb[b], isem[b]).wait()
        pltpu.async_copy(g_hbm.at[srcb[b]], rows[b], gsem[b])

    @pl.loop(0, GROUPS)
    def _(grp):
        base_ch = grp * NBUF
        for b in range(NBUF):
            ch = base_ch + b
            pltpu.make_async_copy(g_hbm.at[srcb[b]], rows[b], gsem[b]).wait()
            pltpu.make_async_copy(dst_hbm.at[pl.ds(base, CHUNK)], dstb[b],
                                  dsem[b]).wait()
            pltpu.async_copy(rows[b], acc.at[dstb[b]], ssem[b], add=True)
            nxt = ch + NBUF

            @pl.when(nxt < N_CHUNKS)
            def _pf_src(b=b, nxt=nxt):
                pltpu.async_copy(src_hbm.at[pl.ds(base + nxt * CHUNK, CHUNK)],
                                 srcb[b], isem[b])
        for b in range(NBUF):
            ch = base_ch + b
            pltpu.make_async_copy(rows[b], acc.at[dstb[b]], ssem[b]).wait()
            nxt = ch + NBUF

            @pl.when(nxt < N_CHUNKS)
            def _refill(b=b, nxt=nxt):
                pltpu.async_copy(dst_hbm.at[pl.ds(base + nxt * CHUNK, CHUNK)],
                                 dstb[b], dsem[b])
                pltpu.make_async_copy(src_hbm.at[pl.ds(base, CHUNK)], srcb[b],
                                      isem[b]).wait()
                pltpu.async_copy(g_hbm.at[srcb[b]], rows[b], gsem[b])

    plsc.subcore_barrier()

    pltpu.sync_copy(acc.at[pl.ds(s * ROWS_PER_TILE, ROWS_PER_TILE)],
                    out_hbm.at[c, pl.ds(s * ROWS_PER_TILE, ROWS_PER_TILE)])


ROW_BLK = 2000


def _tc_head(h, wlT, wcT, bias):
    def body(h_ref, wl_ref, wc_ref, b_ref, g_ref, d_ref):
        hb = h_ref[...]
        g_ref[...] = jnp.dot(hb, wl_ref[...], preferred_element_type=jnp.float32)
        d_ref[...] = jnp.dot(hb, wc_ref[...], preferred_element_type=jnp.float32) + b_ref[...]

    return pl.pallas_call(
        body,
        grid=(N // ROW_BLK,),
        in_specs=[
            pl.BlockSpec((ROW_BLK, D), lambda i: (i, 0)),
            pl.BlockSpec((D, D), lambda i: (0, 0)),
            pl.BlockSpec((D, D), lambda i: (0, 0)),
            pl.BlockSpec((1, D), lambda i: (0, 0)),
        ],
        out_specs=[
            pl.BlockSpec((ROW_BLK, D), lambda i: (i, 0)),
            pl.BlockSpec((ROW_BLK, D), lambda i: (i, 0)),
        ],
        out_shape=[
            jax.ShapeDtypeStruct((N, D), jnp.float32),
            jax.ShapeDtypeStruct((N, D), jnp.float32),
        ],
    )(h, wlT, wcT, bias)


def _tc_mid(p, d, wlT, wcT, bias):
    def body(p_ref, d_ref, wl_ref, wc_ref, b_ref, g_ref, d2_ref):
        hb = jnp.maximum(p_ref[0] + p_ref[1] + d_ref[...], 0.0)
        g_ref[...] = jnp.dot(hb, wl_ref[...], preferred_element_type=jnp.float32)
        d2_ref[...] = jnp.dot(hb, wc_ref[...], preferred_element_type=jnp.float32) + b_ref[...]

    return pl.pallas_call(
        body,
        grid=(N // ROW_BLK,),
        in_specs=[
            pl.BlockSpec((NC, ROW_BLK, D), lambda i: (0, i, 0)),
            pl.BlockSpec((ROW_BLK, D), lambda i: (i, 0)),
            pl.BlockSpec((D, D), lambda i: (0, 0)),
            pl.BlockSpec((D, D), lambda i: (0, 0)),
            pl.BlockSpec((1, D), lambda i: (0, 0)),
        ],
        out_specs=[
            pl.BlockSpec((ROW_BLK, D), lambda i: (i, 0)),
            pl.BlockSpec((ROW_BLK, D), lambda i: (i, 0)),
        ],
        out_shape=[
            jax.ShapeDtypeStruct((N, D), jnp.float32),
            jax.ShapeDtypeStruct((N, D), jnp.float32),
        ],
    )(p, d, wlT, wcT, bias)


def _tc_tail(p, d, owT, ob):
    def body(p_ref, d_ref, ow_ref, ob_ref, o_ref):
        hb = jnp.maximum(p_ref[0] + p_ref[1] + d_ref[...], 0.0)
        o_ref[...] = jnp.dot(hb, ow_ref[...], preferred_element_type=jnp.float32) + ob_ref[...]

    return pl.pallas_call(
        body,
        grid=(N // ROW_BLK,),
        in_specs=[
            pl.BlockSpec((NC, ROW_BLK, D), lambda i: (0, i, 0)),
            pl.BlockSpec((ROW_BLK, D), lambda i: (i, 0)),
            pl.BlockSpec((D, D), lambda i: (0, 0)),
            pl.BlockSpec((1, D), lambda i: (0, 0)),
        ],
        out_specs=pl.BlockSpec((ROW_BLK, D), lambda i: (i, 0)),
        out_shape=jax.ShapeDtypeStruct((N, D), jnp.float32),
    )(p, d, owT, ob)


def _pad_edges(ei):
    pad = jnp.arange(E_PAD - E, dtype=jnp.int32)
    src = jnp.concatenate([ei[1], pad % N])
    dst = jnp.concatenate([ei[0], N + pad % (N_PAD - N)])
    return src, dst


def kernel(x, edge_index_r0, edge_index_r1,
           l0_w0_w, l0_w0_b, l0_wl_w, l0_wl_b, l0_w1_w, l0_w1_b,
           l1_w0_w, l1_w0_b, l1_wl_w, l1_wl_b, l1_w1_w, l1_w1_b,
           out_w, out_b):
    wl1T = l1_wl_w.T
    wc1T = (l1_w0_w + l1_w1_w).T
    b1 = (l1_wl_b + l1_w0_b + l1_w1_b).reshape(1, D)
    wl0T = l0_wl_w.T
    wc0T = (l0_w0_w + l0_w1_w).T
    b0 = (l0_wl_b + l0_w0_b + l0_w1_b).reshape(1, D)
    owT = out_w.T
    ob = out_b.reshape(1, D)

    src1, dst1 = _pad_edges(edge_index_r1)
    src0, dst0 = _pad_edges(edge_index_r0)
    zeros = jnp.zeros((ROWS_PER_TILE, D), jnp.float32)

    g1, d1 = _tc_head(x, wl1T, wc1T, b1)
    p1 = _sc_segment_sum(g1, src1, dst1, zeros)
    g2, d2 = _tc_mid(p1, d1, wl0T, wc0T, b0)
    p2 = _sc_segment_sum(g2, src0, dst0, zeros)
    return _tc_tail(p2, d2, owT, ob)
